# Initial kernel scaffold; baseline (speedup 1.0000x reference)
import jax
import jax.numpy as jnp
from jax.experimental import pallas as pl


def kernel(senders, receivers, node_features, edge_features, eW0, eb0, eW1, eb1, eW2, eb2, eg, ebt, nW0, nb0, nW1, nb1, nW2, nb2, ng, nbt):
    raise NotImplementedError("write your pallas kernel here")


# speedup vs baseline: 2.8471x; 2.8471x over previous
import functools

import jax
import jax.numpy as jnp
from jax import lax
from jax.experimental import pallas as pl
from jax.experimental.pallas import tpu as pltpu
from jax.experimental.pallas import tpu_sc as plsc

N = 10000
E = 160000
H = 256

NC = 2
NS = 16
NW = NC * NS

G_CHUNK = 40
G_NCHUNK = (E // NW) // G_CHUNK

S_PER_TILE = E // NS
S_CHUNK = 80
S_NCHUNK = S_PER_TILE // S_CHUNK

N_PAD = 10240


def _ln(x, g, b):
    mu = jnp.mean(x, axis=-1, keepdims=True)
    xc = x - mu
    var = jnp.mean(xc * xc, axis=-1, keepdims=True)
    return xc * lax.rsqrt(var + 1e-5) * g + b


def _proj_body(nf, w_s, w_r, b0, ps, pr):
    x = nf[...]
    ps[...] = jnp.dot(x, w_s[...], preferred_element_type=jnp.float32)
    pr[...] = jnp.dot(x, w_r[...], preferred_element_type=jnp.float32) + b0[...]


def _edge_body(gsum, ef, w_e, w1, w2, b1, b2, g, bt, new_edge, pre_t):
    e = ef[...]
    x = gsum[...] + jnp.dot(e, w_e[...], preferred_element_type=jnp.float32)
    x = jnp.maximum(x, 0.0)
    x = jnp.dot(x, w1[...], preferred_element_type=jnp.float32) + b1[...]
    x = jnp.maximum(x, 0.0)
    x = jnp.dot(x, w2[...], preferred_element_type=jnp.float32) + b2[...]
    y = _ln(x, g[...], bt[...])
    new_edge[...] = y + e
    pre_t[0] = y[:, :128]
    pre_t[1] = y[:, 128:]


def _node_body(nf, a0, a1, w0, w0lo, w0hi, w1, w2, b0, b1, b2, g, bt, out):
    x0 = nf[...]
    x = (jnp.dot(x0, w0[...], preferred_element_type=jnp.float32)
         + jnp.dot(a0[0], w0lo[...], preferred_element_type=jnp.float32)
         + jnp.dot(a1[0], w0hi[...], preferred_element_type=jnp.float32)
         + b0[...])
    x = jnp.maximum(x, 0.0)
    x = jnp.dot(x, w1[...], preferred_element_type=jnp.float32) + b1[...]
    x = jnp.maximum(x, 0.0)
    x = jnp.dot(x, w2[...], preferred_element_type=jnp.float32) + b2[...]
    out[...] = _ln(x, g[...], bt[...]) + x0


def _full(shape):
    return pl.BlockSpec(shape, lambda i: (0,) * len(shape))


def _rows(bm, w):
    return pl.BlockSpec((bm, w), lambda i: (i, 0))


@functools.cache
def _sc_kernels():
    mesh = plsc.VectorSubcoreMesh(
        core_axis_name="c", subcore_axis_name="s",
        num_cores=NC, num_subcores=NS)

    @functools.partial(
        pl.kernel,
        out_type=jax.ShapeDtypeStruct((NW, G_NCHUNK, G_CHUNK, H), jnp.float32),
        mesh=mesh,
        scratch_types=[
            pltpu.VMEM((G_NCHUNK, G_CHUNK), jnp.int32),
            pltpu.VMEM((G_NCHUNK, G_CHUNK), jnp.int32),
            pltpu.VMEM((G_CHUNK, H), jnp.float32),
            pltpu.VMEM((G_CHUNK, H), jnp.float32),
            pltpu.SemaphoreType.DMA,
            pltpu.SemaphoreType.DMA,
        ],
    )
    def sc_gather(ps_hbm, pr_hbm, sidx_hbm, ridx_hbm, out_hbm,
                  sidx_v, ridx_v, rows_a, rows_b, sem_a, sem_b):
        wid = lax.axis_index("s") * NC + lax.axis_index("c")
        pltpu.sync_copy(sidx_hbm.at[wid], sidx_v)
        pltpu.sync_copy(ridx_hbm.at[wid], ridx_v)

        def chunk(j, carry):
            cp_a = pltpu.async_copy(ps_hbm.at[sidx_v.at[j]], rows_a, sem_a)
            cp_b = pltpu.async_copy(pr_hbm.at[ridx_v.at[j]], rows_b, sem_b)
            cp_a.wait()
            cp_b.wait()

            def add_row(i, c2):
                for k in range(H // 16):
                    sl = pl.ds(k * 16, 16)
                    rows_a[i, sl] = rows_a[i, sl] + rows_b[i, sl]
                return c2

            lax.fori_loop(0, G_CHUNK, add_row, 0, unroll=1)
            pltpu.sync_copy(rows_a, out_hbm.at[wid, j])
            return carry

        lax.fori_loop(0, G_NCHUNK, chunk, 0, unroll=1)

    @functools.partial(
        pl.kernel,
        out_type=jax.ShapeDtypeStruct((NC, N_PAD, 128), jnp.float32),
        mesh=mesh,
        scratch_types=[
            pltpu.VMEM((S_NCHUNK, S_CHUNK), jnp.int32),
            pltpu.VMEM((S_CHUNK, 128), jnp.float32),
            pltpu.VMEM_SHARED((N_PAD, 128), jnp.float32),
        ],
    )
    def sc_scatter(pre_hbm, ridx_hbm, zeros_hbm, out_hbm, ridx_v, rows_v, acc):
        c = lax.axis_index("c")
        s = lax.axis_index("s")
        pltpu.sync_copy(zeros_hbm, acc.at[pl.ds(s * (N_PAD // NS), N_PAD // NS)])
        plsc.subcore_barrier()
        pltpu.sync_copy(ridx_hbm.at[s], ridx_v)

        def chunk(j, carry):
            row0 = s * S_PER_TILE + j * S_CHUNK
            pltpu.sync_copy(pre_hbm.at[c, pl.ds(row0, S_CHUNK)], rows_v)
            pltpu.sync_copy(rows_v, acc.at[ridx_v.at[j]], add=True)
            return carry

        lax.fori_loop(0, S_NCHUNK, chunk, 0, unroll=1)
        plsc.subcore_barrier()
        rpt = N_PAD // NS
        pltpu.sync_copy(acc.at[pl.ds(s * rpt, rpt)],
                        out_hbm.at[c, pl.ds(s * rpt, rpt)])

    return sc_gather, sc_scatter


def _sc_gather(ps, pr, sidx, ridx):
    return _sc_kernels()[0](ps, pr, sidx, ridx)


def _sc_scatter(pre_t, ridx_t, zeros):
    return _sc_kernels()[1](pre_t, ridx_t, zeros)


def kernel(senders, receivers, node_features, edge_features,
           eW0, eb0, eW1, eb1, eW2, eb2, eg, ebt,
           nW0, nb0, nW1, nb1, nW2, nb2, ng, nbt):
    f32 = jnp.float32
    nf = node_features
    ef = edge_features

    eb0r = eb0.reshape(1, H)
    eb1r = eb1.reshape(1, H)
    eb2r = eb2.reshape(1, H)
    egr = eg.reshape(1, H)
    ebtr = ebt.reshape(1, H)
    nb0r = nb0.reshape(1, H)
    nb1r = nb1.reshape(1, H)
    nb2r = nb2.reshape(1, H)
    ngr = ng.reshape(1, H)
    nbtr = nbt.reshape(1, H)

    BN = 2000
    proj_s, proj_r = pl.pallas_call(
        _proj_body,
        grid=(N // BN,),
        in_specs=[_rows(BN, H), _full((H, H)), _full((H, H)), _full((1, H))],
        out_specs=[_rows(BN, H), _rows(BN, H)],
        out_shape=[jax.ShapeDtypeStruct((N, H), f32)] * 2,
    )(nf, eW0[:H], eW0[H:2 * H], eb0r)

    sidx = senders.reshape(NW, G_NCHUNK, G_CHUNK)
    ridx = receivers.reshape(NW, G_NCHUNK, G_CHUNK)
    gsum = _sc_gather(proj_s, proj_r, sidx, ridx).reshape(E, H)

    BE = 2000
    new_edge, pre_t = pl.pallas_call(
        _edge_body,
        grid=(E // BE,),
        in_specs=[_rows(BE, H), _rows(BE, H),
                  _full((H, H)), _full((H, H)), _full((H, H)),
                  _full((1, H)), _full((1, H)), _full((1, H)), _full((1, H))],
        out_specs=[_rows(BE, H),
                   pl.BlockSpec((2, BE, 128), lambda i: (0, i, 0))],
        out_shape=[jax.ShapeDtypeStruct((E, H), f32),
                   jax.ShapeDtypeStruct((2, E, 128), f32)],
    )(gsum, ef, eW0[2 * H:], eW1, eW2, eb1r, eb2r, egr, ebtr)

    ridx_t = receivers.reshape(NS, S_NCHUNK, S_CHUNK)
    zeros = jnp.zeros((N_PAD // NS, 128), f32)
    agg_t = _sc_scatter(pre_t, ridx_t, zeros)[:, :N, :]

    new_node = pl.pallas_call(
        _node_body,
        grid=(N // BN,),
        in_specs=[_rows(BN, H),
                  pl.BlockSpec((1, BN, 128), lambda i: (0, i, 0)),
                  pl.BlockSpec((1, BN, 128), lambda i: (1, i, 0)),
                  _full((H, H)), _full((128, H)), _full((128, H)),
                  _full((H, H)), _full((H, H)),
                  _full((1, H)), _full((1, H)), _full((1, H)),
                  _full((1, H)), _full((1, H))],
        out_specs=_rows(BN, H),
        out_shape=jax.ShapeDtypeStruct((N, H), f32),
    )(nf, agg_t, agg_t, nW0[:H], nW0[H:H + 128], nW0[H + 128:],
      nW1, nW2, nb0r, nb1r, nb2r, ngr, nbtr)

    return (new_node, new_edge)


# speedup vs baseline: 3.9755x; 1.3963x over previous
import functools

import jax
import jax.numpy as jnp
from jax import lax
from jax.experimental import pallas as pl
from jax.experimental.pallas import tpu as pltpu
from jax.experimental.pallas import tpu_sc as plsc

N = 10000
E = 160000
H = 256

NC = 2
NS = 16
NW = NC * NS

G_CHUNK = 40
G_NCHUNK = (E // NW) // G_CHUNK

S_PER_TILE = E // NS
S_CHUNK = 80
S_NCHUNK = S_PER_TILE // S_CHUNK

N_PAD = 10240


def _ln(x, g, b):
    mu = jnp.mean(x, axis=-1, keepdims=True)
    xc = x - mu
    var = jnp.mean(xc * xc, axis=-1, keepdims=True)
    return xc * lax.rsqrt(var + 1e-5) * g + b


def _proj_body(nf, w_s, w_r, b0, ps, pr):
    x = nf[...]
    ps[...] = jnp.dot(x, w_s[...], preferred_element_type=jnp.float32)
    pr[...] = jnp.dot(x, w_r[...], preferred_element_type=jnp.float32) + b0[...]


def _edge_body(gsum, ef, w_e, w1, w2, b1, b2, g, bt, new_edge, pre_t):
    e = ef[...]
    x = gsum[...] + jnp.dot(e, w_e[...], preferred_element_type=jnp.float32)
    x = jnp.maximum(x, 0.0)
    x = jnp.dot(x, w1[...], preferred_element_type=jnp.float32) + b1[...]
    x = jnp.maximum(x, 0.0)
    x = jnp.dot(x, w2[...], preferred_element_type=jnp.float32) + b2[...]
    y = _ln(x, g[...], bt[...])
    new_edge[...] = y + e
    pre_t[0] = y[:, :128]
    pre_t[1] = y[:, 128:]


def _node_body(nf, a0, a1, w0, w0lo, w0hi, w1, w2, b0, b1, b2, g, bt, out):
    x0 = nf[...]
    x = (jnp.dot(x0, w0[...], preferred_element_type=jnp.float32)
         + jnp.dot(a0[0], w0lo[...], preferred_element_type=jnp.float32)
         + jnp.dot(a1[0], w0hi[...], preferred_element_type=jnp.float32)
         + b0[...])
    x = jnp.maximum(x, 0.0)
    x = jnp.dot(x, w1[...], preferred_element_type=jnp.float32) + b1[...]
    x = jnp.maximum(x, 0.0)
    x = jnp.dot(x, w2[...], preferred_element_type=jnp.float32) + b2[...]
    out[...] = _ln(x, g[...], bt[...]) + x0


def _full(shape):
    return pl.BlockSpec(shape, lambda i: (0,) * len(shape))


def _rows(bm, w):
    return pl.BlockSpec((bm, w), lambda i: (i, 0))


@functools.cache
def _sc_kernels():
    mesh = plsc.VectorSubcoreMesh(
        core_axis_name="c", subcore_axis_name="s",
        num_cores=NC, num_subcores=NS)

    @functools.partial(
        pl.kernel,
        out_type=jax.ShapeDtypeStruct((NW, G_NCHUNK, G_CHUNK, H), jnp.float32),
        mesh=mesh,
        scratch_types=[
            pltpu.VMEM((G_NCHUNK, G_CHUNK), jnp.int32),
            pltpu.VMEM((G_NCHUNK, G_CHUNK), jnp.int32),
            [pltpu.VMEM((G_CHUNK, H), jnp.float32)] * 2,
            [pltpu.VMEM((G_CHUNK, H), jnp.float32)] * 2,
            [pltpu.SemaphoreType.DMA] * 2,
            [pltpu.SemaphoreType.DMA] * 2,
            [pltpu.SemaphoreType.DMA] * 2,
        ],
    )
    def sc_gather(ps_hbm, pr_hbm, sidx_hbm, ridx_hbm, out_hbm,
                  sidx_v, ridx_v, rows_a, rows_b, sem_a, sem_b, sem_w):
        wid = lax.axis_index("s") * NC + lax.axis_index("c")
        pltpu.sync_copy(sidx_hbm.at[wid], sidx_v)
        pltpu.sync_copy(ridx_hbm.at[wid], ridx_v)

        def start(j, b):
            pltpu.async_copy(ps_hbm.at[sidx_v.at[j]], rows_a[b], sem_a[b])
            pltpu.async_copy(pr_hbm.at[ridx_v.at[j]], rows_b[b], sem_b[b])

        def process(j, b):
            pltpu.make_async_copy(ps_hbm.at[sidx_v.at[j]], rows_a[b],
                                  sem_a[b]).wait()
            pltpu.make_async_copy(pr_hbm.at[ridx_v.at[j]], rows_b[b],
                                  sem_b[b]).wait()

            def add_row(i, c2):
                for k in range(H // 16):
                    sl = pl.ds(k * 16, 16)
                    rows_a[b][i, sl] = rows_a[b][i, sl] + rows_b[b][i, sl]
                return c2

            lax.fori_loop(0, G_CHUNK, add_row, 0, unroll=1)
            pltpu.async_copy(rows_a[b], out_hbm.at[wid, j], sem_w[b])

        def wait_write(j, b):
            pltpu.make_async_copy(rows_a[b], out_hbm.at[wid, j],
                                  sem_w[b]).wait()

        start(0, 0)

        def pair(t, carry):
            for b in range(2):
                j = 2 * t + b

                @pl.when(j > 0)
                def _():
                    wait_write(j - 1, 1 - b)

                start(j + 1, 1 - b)
                process(j, b)
            return carry

        lax.fori_loop(0, (G_NCHUNK - 1) // 2, pair, 0, unroll=1)
        wait_write(G_NCHUNK - 2, 1)
        process(G_NCHUNK - 1, 0)
        wait_write(G_NCHUNK - 1, 0)

    @functools.partial(
        pl.kernel,
        out_type=jax.ShapeDtypeStruct((NC, N_PAD, 128), jnp.float32),
        mesh=mesh,
        scratch_types=[
            pltpu.VMEM((S_NCHUNK, S_CHUNK), jnp.int32),
            [pltpu.VMEM((S_CHUNK, 128), jnp.float32)] * 2,
            pltpu.VMEM_SHARED((N_PAD, 128), jnp.float32),
            [pltpu.SemaphoreType.DMA] * 2,
            [pltpu.SemaphoreType.DMA] * 2,
        ],
    )
    def sc_scatter(pre_hbm, ridx_hbm, zeros_hbm, out_hbm,
                   ridx_v, rows_v, acc, sem_l, sem_s):
        c = lax.axis_index("c")
        s = lax.axis_index("s")
        pltpu.sync_copy(zeros_hbm, acc.at[pl.ds(s * (N_PAD // NS), N_PAD // NS)])
        plsc.subcore_barrier()
        pltpu.sync_copy(ridx_hbm.at[s], ridx_v)

        def _src(j):
            return pre_hbm.at[c, pl.ds(s * S_PER_TILE + j * S_CHUNK, S_CHUNK)]

        def start_load(j, b):
            pltpu.async_copy(_src(j), rows_v[b], sem_l[b])

        def start_scatter(j, b):
            pltpu.async_copy(rows_v[b], acc.at[ridx_v.at[j]], sem_s[b],
                             add=True)

        def wait_load(j, b):
            pltpu.make_async_copy(_src(j), rows_v[b], sem_l[b]).wait()

        def wait_scatter(j, b):
            pltpu.make_async_copy(rows_v[b], acc.at[ridx_v.at[j]],
                                  sem_s[b]).wait()

        start_load(0, 0)

        def pair(t, carry):
            for b in range(2):
                j = 2 * t + b

                @pl.when(j > 0)
                def _():
                    wait_scatter(j - 1, 1 - b)

                start_load(j + 1, 1 - b)
                wait_load(j, b)
                start_scatter(j, b)
            return carry

        lax.fori_loop(0, (S_NCHUNK - 1) // 2, pair, 0, unroll=1)
        wait_scatter(S_NCHUNK - 2, 1)
        wait_load(S_NCHUNK - 1, 0)
        start_scatter(S_NCHUNK - 1, 0)
        wait_scatter(S_NCHUNK - 1, 0)
        plsc.subcore_barrier()
        rpt = N_PAD // NS
        pltpu.sync_copy(acc.at[pl.ds(s * rpt, rpt)],
                        out_hbm.at[c, pl.ds(s * rpt, rpt)])

    return sc_gather, sc_scatter


def _sc_gather(ps, pr, sidx, ridx):
    return _sc_kernels()[0](ps, pr, sidx, ridx)


def _sc_scatter(pre_t, ridx_t, zeros):
    return _sc_kernels()[1](pre_t, ridx_t, zeros)


def kernel(senders, receivers, node_features, edge_features,
           eW0, eb0, eW1, eb1, eW2, eb2, eg, ebt,
           nW0, nb0, nW1, nb1, nW2, nb2, ng, nbt):
    f32 = jnp.float32
    nf = node_features
    ef = edge_features

    eb0r = eb0.reshape(1, H)
    eb1r = eb1.reshape(1, H)
    eb2r = eb2.reshape(1, H)
    egr = eg.reshape(1, H)
    ebtr = ebt.reshape(1, H)
    nb0r = nb0.reshape(1, H)
    nb1r = nb1.reshape(1, H)
    nb2r = nb2.reshape(1, H)
    ngr = ng.reshape(1, H)
    nbtr = nbt.reshape(1, H)

    BN = 2000
    proj_s, proj_r = pl.pallas_call(
        _proj_body,
        grid=(N // BN,),
        in_specs=[_rows(BN, H), _full((H, H)), _full((H, H)), _full((1, H))],
        out_specs=[_rows(BN, H), _rows(BN, H)],
        out_shape=[jax.ShapeDtypeStruct((N, H), f32)] * 2,
    )(nf, eW0[:H], eW0[H:2 * H], eb0r)

    sidx = senders.reshape(NW, G_NCHUNK, G_CHUNK)
    ridx = receivers.reshape(NW, G_NCHUNK, G_CHUNK)
    gsum = _sc_gather(proj_s, proj_r, sidx, ridx).reshape(E, H)

    BE = 2000
    new_edge, pre_t = pl.pallas_call(
        _edge_body,
        grid=(E // BE,),
        in_specs=[_rows(BE, H), _rows(BE, H),
                  _full((H, H)), _full((H, H)), _full((H, H)),
                  _full((1, H)), _full((1, H)), _full((1, H)), _full((1, H))],
        out_specs=[_rows(BE, H),
                   pl.BlockSpec((2, BE, 128), lambda i: (0, i, 0))],
        out_shape=[jax.ShapeDtypeStruct((E, H), f32),
                   jax.ShapeDtypeStruct((2, E, 128), f32)],
    )(gsum, ef, eW0[2 * H:], eW1, eW2, eb1r, eb2r, egr, ebtr)

    ridx_t = receivers.reshape(NS, S_NCHUNK, S_CHUNK)
    zeros = jnp.zeros((N_PAD // NS, 128), f32)
    agg_t = _sc_scatter(pre_t, ridx_t, zeros)[:, :N, :]

    new_node = pl.pallas_call(
        _node_body,
        grid=(N // BN,),
        in_specs=[_rows(BN, H),
                  pl.BlockSpec((1, BN, 128), lambda i: (0, i, 0)),
                  pl.BlockSpec((1, BN, 128), lambda i: (1, i, 0)),
                  _full((H, H)), _full((128, H)), _full((128, H)),
                  _full((H, H)), _full((H, H)),
                  _full((1, H)), _full((1, H)), _full((1, H)),
                  _full((1, H)), _full((1, H))],
        out_specs=_rows(BN, H),
        out_shape=jax.ShapeDtypeStruct((N, H), f32),
    )(nf, agg_t, agg_t, nW0[:H], nW0[H:H + 128], nW0[H + 128:],
      nW1, nW2, nb0r, nb1r, nb2r, ngr, nbtr)

    return (new_node, new_edge)
